# baseline (speedup 1.0000x reference)
import jax
import jax.numpy as jnp
from jax import lax
from jax.experimental import pallas as pl
from jax.experimental.pallas import tpu as pltpu
from jax.experimental.pallas import tpu_sc as plsc

_NUM_COLLISIONS = 4
_EMBED = 64
_LANES = 16
_NC, _NS = 2, 16
_NW = _NC * _NS
_IDX_CHUNK = 128


def _qr_body(idx_hbm, q_hbm, r_hbm, out_hbm,
             idx_v, qidx_v, rem_v, qrows, rrows, qsem, rsem):
    b_per_w = idx_v.shape[0]
    n16 = b_per_w // _LANES
    n128 = b_per_w // _IDX_CHUNK
    wid = lax.axis_index("s") * _NC + lax.axis_index("c")
    base = wid * b_per_w

    pltpu.sync_copy(idx_hbm.at[pl.ds(base, b_per_w)], idx_v)

    for j in range(n16):
        v = idx_v[pl.ds(j * _LANES, _LANES)]
        row = j // (_IDX_CHUNK // _LANES)
        col = (j % (_IDX_CHUNK // _LANES)) * _LANES
        qidx_v[row, pl.ds(col, _LANES)] = v >> 2
        rem_v[row, pl.ds(col, _LANES)] = v & (_NUM_COLLISIONS - 1)

    copies = []
    for j in range(n128):
        dst = pl.ds(j * _IDX_CHUNK, _IDX_CHUNK)
        copies.append(pltpu.async_copy(q_hbm.at[qidx_v.at[j]], qrows.at[dst], qsem))
        copies.append(pltpu.async_copy(r_hbm.at[rem_v.at[j]], rrows.at[dst], rsem))
    for c in copies:
        c.wait()

    def mul_body(i, carry):
        for ch in range(_EMBED // _LANES):
            sl = pl.ds(ch * _LANES, _LANES)
            qrows[i, sl] = qrows[i, sl] * rrows[i, sl]
        return carry
    lax.fori_loop(0, b_per_w, mul_body, 0)

    pltpu.sync_copy(qrows, out_hbm.at[pl.ds(base, b_per_w)])


def kernel(inputs, q_table, r_table):
    batch = inputs.shape[0]
    assert batch % (_NW * _IDX_CHUNK) == 0
    b_per_w = batch // _NW
    mesh = plsc.VectorSubcoreMesh(core_axis_name="c", subcore_axis_name="s")
    k = pl.kernel(
        _qr_body,
        out_type=jax.ShapeDtypeStruct((batch, _EMBED), jnp.float32),
        mesh=mesh,
        scratch_types=[
            pltpu.VMEM((b_per_w,), jnp.int32),
            pltpu.VMEM((b_per_w // _IDX_CHUNK, _IDX_CHUNK), jnp.int32),
            pltpu.VMEM((b_per_w // _IDX_CHUNK, _IDX_CHUNK), jnp.int32),
            pltpu.VMEM((b_per_w, _EMBED), jnp.float32),
            pltpu.VMEM((b_per_w, _EMBED), jnp.float32),
            pltpu.SemaphoreType.DMA,
            pltpu.SemaphoreType.DMA,
        ],
        compiler_params=pltpu.CompilerParams(use_tc_tiling_on_sc=False),
    )
    return k(inputs.astype(jnp.int32), q_table, r_table)


# speedup vs baseline: 1.7688x; 1.7688x over previous
import jax
import jax.numpy as jnp
from jax import lax
from jax.experimental import pallas as pl
from jax.experimental.pallas import tpu as pltpu
from jax.experimental.pallas import tpu_sc as plsc

_NUM_COLLISIONS = 4
_EMBED = 64
_LANES = 16
_NC, _NS = 2, 16
_NW = _NC * _NS
_IDX_CHUNK = 128


def _qr_body(idx_hbm, q_hbm, r_hbm, out_hbm,
             idx_v, qidx_v, r_v, qrows, qsem):
    b_per_w = idx_v.shape[0]
    n16 = b_per_w // _LANES
    n128 = b_per_w // _IDX_CHUNK
    wid = lax.axis_index("s") * _NC + lax.axis_index("c")
    base = wid * b_per_w

    pltpu.sync_copy(idx_hbm.at[pl.ds(base, b_per_w)], idx_v)
    pltpu.sync_copy(r_hbm, r_v)

    for j in range(n16):
        v = idx_v[pl.ds(j * _LANES, _LANES)]
        row = j // (_IDX_CHUNK // _LANES)
        col = (j % (_IDX_CHUNK // _LANES)) * _LANES
        qidx_v[row, pl.ds(col, _LANES)] = v >> 2

    copies = []
    for j in range(n128):
        dst = pl.ds(j * _IDX_CHUNK, _IDX_CHUNK)
        copies.append(pltpu.async_copy(q_hbm.at[qidx_v.at[j]], qrows.at[dst], qsem))
    for c in copies:
        c.wait()

    def mul_body(g, carry):
        remv = idx_v[pl.ds(g * _LANES, _LANES)] & (_NUM_COLLISIONS - 1)
        for j in range(_LANES):
            rem = remv[j]
            i = g * _LANES + j
            for ch in range(_EMBED // _LANES):
                sl = pl.ds(ch * _LANES, _LANES)
                qrows[i, sl] = qrows[i, sl] * r_v[rem, sl]
        return carry
    lax.fori_loop(0, n16, mul_body, 0)

    pltpu.sync_copy(qrows, out_hbm.at[pl.ds(base, b_per_w)])


def kernel(inputs, q_table, r_table):
    batch = inputs.shape[0]
    assert batch % (_NW * _IDX_CHUNK) == 0
    b_per_w = batch // _NW
    mesh = plsc.VectorSubcoreMesh(core_axis_name="c", subcore_axis_name="s")
    k = pl.kernel(
        _qr_body,
        out_type=jax.ShapeDtypeStruct((batch, _EMBED), jnp.float32),
        mesh=mesh,
        scratch_types=[
            pltpu.VMEM((b_per_w,), jnp.int32),
            pltpu.VMEM((b_per_w // _IDX_CHUNK, _IDX_CHUNK), jnp.int32),
            pltpu.VMEM((_NUM_COLLISIONS, _EMBED), jnp.float32),
            pltpu.VMEM((b_per_w, _EMBED), jnp.float32),
            pltpu.SemaphoreType.DMA,
        ],
        compiler_params=pltpu.CompilerParams(use_tc_tiling_on_sc=False),
    )
    return k(inputs.astype(jnp.int32), q_table, r_table)


# speedup vs baseline: 1.8067x; 1.0214x over previous
import jax
import jax.numpy as jnp
from jax import lax
from jax.experimental import pallas as pl
from jax.experimental.pallas import tpu as pltpu
from jax.experimental.pallas import tpu_sc as plsc

_NUM_COLLISIONS = 4
_EMBED = 64
_PAIR = 2 * _EMBED
_LANES = 16
_NC, _NS = 2, 16
_NW = _NC * _NS
_IDX_CHUNK = 128


def _qr_body(idx_hbm, qp_hbm, r_hbm, out_hbm,
             idx_v, pidx_v, r_v, qprows, qsem):
    b_per_w = idx_v.shape[0]
    n16 = b_per_w // _LANES
    n128 = b_per_w // _IDX_CHUNK
    wid = lax.axis_index("s") * _NC + lax.axis_index("c")
    base = wid * b_per_w

    pltpu.sync_copy(idx_hbm.at[pl.ds(base, b_per_w)], idx_v)
    pltpu.sync_copy(r_hbm, r_v)

    for j in range(n16):
        v = idx_v[pl.ds(j * _LANES, _LANES)]
        row = j // (_IDX_CHUNK // _LANES)
        col = (j % (_IDX_CHUNK // _LANES)) * _LANES
        pidx_v[row, pl.ds(col, _LANES)] = v >> 3

    copies = []
    for j in range(n128):
        dst = pl.ds(j * _IDX_CHUNK, _IDX_CHUNK)
        copies.append(pltpu.async_copy(qp_hbm.at[pidx_v.at[j]], qprows.at[dst], qsem))
    for c in copies:
        c.wait()

    def mul_body(g, carry):
        iv = idx_v[pl.ds(g * _LANES, _LANES)]
        remv = iv & (_NUM_COLLISIONS - 1)
        holv = iv & (2 * _NUM_COLLISIONS - 1)
        for j in range(_LANES):
            rem = remv[j]
            hoff = (holv[j] >> 2) * _EMBED
            i = g * _LANES + j
            for ch in range(_EMBED // _LANES):
                off = ch * _LANES
                qprows[i, pl.ds(off, _LANES)] = (
                    qprows[i, pl.ds(hoff + off, _LANES)] * r_v[rem, pl.ds(off, _LANES)]
                )
        return carry
    lax.fori_loop(0, n16, mul_body, 0)

    pltpu.sync_copy(qprows, out_hbm.at[pl.ds(base, b_per_w)])


def kernel(inputs, q_table, r_table):
    batch = inputs.shape[0]
    assert batch % (_NW * _IDX_CHUNK) == 0
    b_per_w = batch // _NW
    q_pairs = jnp.reshape(q_table, (q_table.shape[0] // 2, _PAIR))
    mesh = plsc.VectorSubcoreMesh(core_axis_name="c", subcore_axis_name="s")
    k = pl.kernel(
        _qr_body,
        out_type=jax.ShapeDtypeStruct((batch, _PAIR), jnp.float32),
        mesh=mesh,
        scratch_types=[
            pltpu.VMEM((b_per_w,), jnp.int32),
            pltpu.VMEM((b_per_w // _IDX_CHUNK, _IDX_CHUNK), jnp.int32),
            pltpu.VMEM((_NUM_COLLISIONS, _EMBED), jnp.float32),
            pltpu.VMEM((b_per_w, _PAIR), jnp.float32),
            pltpu.SemaphoreType.DMA,
        ],
    )
    return k(inputs.astype(jnp.int32), q_pairs, r_table)[:, :_EMBED]
